# baseline (speedup 1.0000x reference)
import jax
import jax.numpy as jnp
from jax import lax
from jax.experimental import pallas as pl
from jax.experimental.pallas import tpu as pltpu

_B = 4096
_I = 1024
_BR = 512
_N = 100000
_NROW = 896


def _irt_body(sids_ref, iids_ref, ahi_ref, alo_ref, dhi_ref, dlo_ref,
              out_ref, idf_scr):
    i = pl.program_id(0)

    @pl.when(i == 0)
    def _gather_items():
        iid = iids_ref[...]
        ir = iid >> 7
        ic = (iid & 127)[:, None]
        oh = (lax.broadcasted_iota(jnp.int32, (_I, _NROW), 1)
              == lax.broadcast_in_dim(ir, (_I, _NROW), (0,))
              ).astype(jnp.bfloat16)
        rows_hi = lax.dot(oh, dhi_ref[...], preferred_element_type=jnp.float32)
        rows_lo = lax.dot(oh, dlo_ref[...], preferred_element_type=jnp.float32)
        idf = (jnp.take_along_axis(rows_hi, ic, axis=1, mode="promise_in_bounds")
               + jnp.take_along_axis(rows_lo, ic, axis=1, mode="promise_in_bounds"))
        idf_scr[...] = idf.reshape(1, _I)

    sid = sids_ref[...]
    sr = sid >> 7
    sc = (sid & 127)[:, None]
    oh = (lax.broadcasted_iota(jnp.int32, (_BR, _NROW), 1)
          == lax.broadcast_in_dim(sr, (_BR, _NROW), (0,))
          ).astype(jnp.bfloat16)
    rows_hi = lax.dot(oh, ahi_ref[...], preferred_element_type=jnp.float32)
    rows_lo = lax.dot(oh, alo_ref[...], preferred_element_type=jnp.float32)
    sa = (jnp.take_along_axis(rows_hi, sc, axis=1, mode="promise_in_bounds")
          + jnp.take_along_axis(rows_lo, sc, axis=1, mode="promise_in_bounds"))
    sa_col = jnp.broadcast_to(sa, (_BR, _I))
    idf_row = lax.broadcast_in_dim(idf_scr[0, :], (_BR, _I), (1,))
    out_ref[...] = sa_col - idf_row


def _hi_lo(table):
    t2 = jnp.pad(table, (0, _NROW * 128 - _N)).reshape(_NROW, 128)
    hi = t2.astype(jnp.bfloat16)
    lo = (t2 - hi.astype(jnp.float32)).astype(jnp.bfloat16)
    return hi, lo


@jax.jit
def kernel(student_ids, item_ids, student_ability, item_difficulty):
    sids = student_ids.astype(jnp.int32)
    iids = item_ids.astype(jnp.int32)
    ahi, alo = _hi_lo(student_ability)
    dhi, dlo = _hi_lo(item_difficulty)
    out = pl.pallas_call(
        _irt_body,
        grid=(_B // _BR,),
        in_specs=[
            pl.BlockSpec((_BR,), lambda i: (i,)),
            pl.BlockSpec((_I,), lambda i: (0,)),
            pl.BlockSpec((_NROW, 128), lambda i: (0, 0)),
            pl.BlockSpec((_NROW, 128), lambda i: (0, 0)),
            pl.BlockSpec((_NROW, 128), lambda i: (0, 0)),
            pl.BlockSpec((_NROW, 128), lambda i: (0, 0)),
        ],
        out_specs=pl.BlockSpec((_BR, _I), lambda i: (i, 0)),
        out_shape=jax.ShapeDtypeStruct((_B, _I), jnp.float32),
        scratch_shapes=[pltpu.VMEM((1, _I), jnp.float32)],
    )(sids, iids, ahi, alo, dhi, dlo)
    return out
